# baseline (speedup 1.0000x reference)
import functools

import jax
import jax.numpy as jnp
from jax import lax
from jax.experimental import pallas as pl
from jax.experimental.pallas import tpu as pltpu
from jax.experimental.pallas import tpu_sc as plsc

_N = 100
_NP = 128
_E = 6400
_DO = 60


def _sc_edge_counts_body(edge_hbm, zeros_hbm, out_hbm, src_v, dst_v, m_v):
    cid = lax.axis_index("c")
    sid = lax.axis_index("s")

    @pl.when(jnp.logical_and(cid == 0, sid == 0))
    def _():
        pltpu.sync_copy(edge_hbm.at[0], src_v)
        pltpu.sync_copy(edge_hbm.at[1], dst_v)
        pltpu.sync_copy(zeros_hbm, m_v)

        def body(i, carry):
            s = src_v[pl.ds(i * 16, 16)]
            d = dst_v[pl.ds(i * 16, 16)]
            ew = jnp.where(s == d, jnp.float32(0.0), jnp.float32(1.0))
            plsc.addupdate_scatter(m_v, [s * _NP + d], ew)
            return carry

        lax.fori_loop(0, _E // 16, body, 0)
        pltpu.sync_copy(m_v, out_hbm)


@functools.cache
def _sc_edge_counts():
    mesh = plsc.VectorSubcoreMesh(core_axis_name="c", subcore_axis_name="s")
    return pl.kernel(
        _sc_edge_counts_body,
        mesh=mesh,
        out_type=jax.ShapeDtypeStruct((_NP * _NP,), jnp.float32),
        scratch_types=[
            pltpu.VMEM((_E,), jnp.int32),
            pltpu.VMEM((_E,), jnp.int32),
            pltpu.VMEM((_NP * _NP,), jnp.float32),
        ],
        compiler_params=pltpu.CompilerParams(needs_layout_passes=False),
    )


def _tc_body(sf_ref, m_ref, wa_ref, ba_ref, wc_ref, bc_ref,
             wfca_ref, bfca_ref, wfcc_ref, bfcc_ref, scal_ref,
             log_ref, val_ref, ga_s, gc_s):
    x = sf_ref[0]
    m = m_ref[0:_N, :]
    deg = jnp.sum(m, axis=1, keepdims=True)
    dis = jnp.where(deg > 0, 1.0 / jnp.sqrt(jnp.maximum(deg, 1.0)), 0.0)

    def prop(h):
        z = lax.dot_general(m, dis * h, (((0,), (0,)), ((), ())),
                            preferred_element_type=jnp.float32)
        return -dis * z[0:_N, :]

    def head(w_ref, b_ref, g_s):
        a0 = jnp.dot(x, w_ref[0], preferred_element_type=jnp.float32)
        a1 = jnp.dot(x, w_ref[1], preferred_element_type=jnp.float32)
        a2 = jnp.dot(x, w_ref[2], preferred_element_type=jnp.float32)
        conv = a0 - a2 + prop(a1 + 2.0 * prop(a2)) + b_ref[...]
        g_s[...] = jnp.tanh(conv)

    head(wa_ref, ba_ref, ga_s)
    head(wc_ref, bc_ref, gc_s)

    def fc_step(n, carry):
        acc_a, acc_c = carry
        wa_n = wfca_ref[pl.ds(n * _DO, _DO), :]
        wc_n = wfcc_ref[pl.ds(n * _DO, _DO), :]
        ga_n = ga_s[pl.ds(n, 1), :]
        gc_n = gc_s[pl.ds(n, 1), :]
        acc_a = acc_a + jnp.dot(ga_n, wa_n, preferred_element_type=jnp.float32)
        acc_c = acc_c + jnp.dot(gc_n, wc_n, preferred_element_type=jnp.float32)
        return acc_a, acc_c

    acc_a = jnp.zeros((1, 100), jnp.float32)
    acc_c = jnp.zeros((1, 1), jnp.float32)
    acc_a, acc_c = lax.fori_loop(0, _N, fc_step, (acc_a, acc_c))

    scal = scal_ref[...]
    tail_a = jnp.dot(scal, wfca_ref[_N * _DO:, :],
                     preferred_element_type=jnp.float32)
    tail_c = jnp.dot(scal, wfcc_ref[_N * _DO:, :],
                     preferred_element_type=jnp.float32)
    log_ref[...] = acc_a + tail_a + bfca_ref[...]
    val_ref[...] = acc_c + tail_c + bfcc_ref[...]


def _tc_call(sf, m, wa, ba, wc, bc, wfca, bfca, wfcc, bfcc, scal):
    return pl.pallas_call(
        _tc_body,
        out_shape=(
            jax.ShapeDtypeStruct((1, 100), jnp.float32),
            jax.ShapeDtypeStruct((1, 1), jnp.float32),
        ),
        scratch_shapes=[
            pltpu.VMEM((_N, _DO), jnp.float32),
            pltpu.VMEM((_N, _DO), jnp.float32),
        ],
    )(sf, m, wa, ba, wc, bc, wfca, bfca, wfcc, bfcc, scal)


def kernel(substrate_features, edge_index, v_cpu_demand_t, v_bw_demand_t,
           num_pending_v_nodes_t, W_actor_conv, b_actor_conv, W_critic_conv,
           b_critic_conv, W_actor_fc, b_actor_fc, W_critic_fc, b_critic_fc):
    zeros = jnp.zeros((_NP * _NP,), jnp.float32)
    m = _sc_edge_counts()(edge_index, zeros).reshape(_NP, _NP)

    scal = jnp.concatenate(
        [v_cpu_demand_t, v_bw_demand_t, num_pending_v_nodes_t])[None, :]

    logits, values = _tc_call(
        substrate_features, m,
        W_actor_conv, b_actor_conv[None, :],
        W_critic_conv, b_critic_conv[None, :],
        W_actor_fc, b_actor_fc[None, :],
        W_critic_fc, b_critic_fc[None, :],
        scal,
    )
    return (logits, values)


# speedup vs baseline: 1.2318x; 1.2318x over previous
import functools

import jax
import jax.numpy as jnp
from jax import lax
from jax.experimental import pallas as pl
from jax.experimental.pallas import tpu as pltpu
from jax.experimental.pallas import tpu_sc as plsc

_N = 100
_NP = 128
_E = 6400
_DO = 60
_NW = 10
_EPW = _E // _NW
_HW = _NP * _NP


def _sc_edge_counts_body(edge_hbm, zeros_hbm, out_hbm, src_v, dst_v, m_v):
    cid = lax.axis_index("c")
    sid = lax.axis_index("s")
    wid = sid * 2 + cid

    @pl.when(wid < _NW)
    def _():
        base = wid * _EPW
        pltpu.sync_copy(edge_hbm.at[0].at[pl.ds(base, _EPW)], src_v)
        pltpu.sync_copy(edge_hbm.at[1].at[pl.ds(base, _EPW)], dst_v)
        pltpu.sync_copy(zeros_hbm, m_v)

        def body(i, carry):
            s = src_v[pl.ds(i * 16, 16)]
            d = dst_v[pl.ds(i * 16, 16)]
            ew = jnp.where(s == d, jnp.float32(0.0), jnp.float32(1.0))
            plsc.addupdate_scatter(m_v, [s * _NP + d], ew)
            return carry

        lax.fori_loop(0, _EPW // 16, body, 0)
        pltpu.sync_copy(m_v, out_hbm.at[pl.ds(wid * _HW, _HW)])


@functools.cache
def _sc_edge_counts():
    mesh = plsc.VectorSubcoreMesh(core_axis_name="c", subcore_axis_name="s")
    return pl.kernel(
        _sc_edge_counts_body,
        mesh=mesh,
        out_type=jax.ShapeDtypeStruct((_NW * _HW,), jnp.float32),
        scratch_types=[
            pltpu.VMEM((_EPW,), jnp.int32),
            pltpu.VMEM((_EPW,), jnp.int32),
            pltpu.VMEM((_HW,), jnp.float32),
        ],
        compiler_params=pltpu.CompilerParams(needs_layout_passes=False),
    )


def _conv_body(sf_ref, m_ref, wa_ref, ba_ref, wc_ref, bc_ref, ga_ref, gc_ref):
    x = sf_ref[0]
    m = m_ref[0:_N, :]
    for w in range(1, _NW):
        m = m + m_ref[w * _NP:w * _NP + _N, :]
    deg = jnp.sum(m, axis=1, keepdims=True)
    dis = jnp.where(deg > 0, 1.0 / jnp.sqrt(jnp.maximum(deg, 1.0)), 0.0)

    def prop(h):
        z = lax.dot_general(m, dis * h, (((0,), (0,)), ((), ())),
                            preferred_element_type=jnp.float32,
                            precision=lax.Precision.HIGHEST)
        return -dis * z[0:_N, :]

    tx1 = prop(x)
    tx2 = 2.0 * prop(tx1) - x

    def head(w_ref, b_ref, g_ref):
        conv = (jnp.dot(x, w_ref[0], preferred_element_type=jnp.float32)
                + jnp.dot(tx1, w_ref[1], preferred_element_type=jnp.float32)
                + jnp.dot(tx2, w_ref[2], preferred_element_type=jnp.float32)
                + b_ref[...])
        g_ref[...] = jnp.tanh(conv)

    head(wa_ref, ba_ref, ga_ref)
    head(wc_ref, bc_ref, gc_ref)


def _conv_call(sf, m, wa, ba, wc, bc):
    return pl.pallas_call(
        _conv_body,
        out_shape=(
            jax.ShapeDtypeStruct((_N, _DO), jnp.float32),
            jax.ShapeDtypeStruct((_N, _DO), jnp.float32),
        ),
    )(sf, m, wa, ba, wc, bc)


def _fc_body(ca_ref, cc_ref, wa_ref, ba_ref, wc_ref, bc_ref, log_ref, val_ref):
    log_ref[...] = (
        jnp.dot(ca_ref[...], wa_ref[...], preferred_element_type=jnp.float32)
        + ba_ref[...]
    )
    val_ref[...] = (
        jnp.dot(cc_ref[...], wc_ref[...], preferred_element_type=jnp.float32,
                precision=lax.Precision.HIGHEST)
        + bc_ref[...]
    )


def _fc_call(cat_a, cat_c, wa, ba, wc, bc):
    return pl.pallas_call(
        _fc_body,
        out_shape=(
            jax.ShapeDtypeStruct((1, 100), jnp.float32),
            jax.ShapeDtypeStruct((1, 1), jnp.float32),
        ),
    )(cat_a, cat_c, wa, ba, wc, bc)


def kernel(substrate_features, edge_index, v_cpu_demand_t, v_bw_demand_t,
           num_pending_v_nodes_t, W_actor_conv, b_actor_conv, W_critic_conv,
           b_critic_conv, W_actor_fc, b_actor_fc, W_critic_fc, b_critic_fc):
    zeros = jnp.zeros((_HW,), jnp.float32)
    m = _sc_edge_counts()(edge_index, zeros).reshape(_NW * _NP, _NP)

    ga, gc = _conv_call(
        substrate_features, m,
        W_actor_conv, b_actor_conv[None, :],
        W_critic_conv, b_critic_conv[None, :],
    )

    scal = [v_cpu_demand_t[None, :], v_bw_demand_t[None, :],
            num_pending_v_nodes_t[None, :]]
    cat_a = jnp.concatenate([ga.reshape(1, _N * _DO)] + scal, axis=1)
    cat_c = jnp.concatenate([gc.reshape(1, _N * _DO)] + scal, axis=1)

    logits, values = _fc_call(
        cat_a, cat_c,
        W_actor_fc, b_actor_fc[None, :],
        W_critic_fc, b_critic_fc[None, :],
    )
    return (logits, values)
